# baseline (speedup 1.0000x reference)
import functools

import jax
import jax.numpy as jnp
from jax import lax
from jax.experimental import pallas as pl
from jax.experimental.pallas import tpu as pltpu
from jax.experimental.pallas import tpu_sc as plsc

N = 10000
E = 320000
D = 128
HID = 256

NC = 2
NS = 16
EPW = E // (NC * NS)
ROWS_PW = 624
ROWS_TAIL = N - NS * ROWS_PW
K = 80
NCH = EPW // K
NB = 5

_sc_mesh = plsc.VectorSubcoreMesh(
    core_axis_name="c", subcore_axis_name="s", num_cores=NC, num_subcores=NS)


def _prop_body(g_hbm, src_hbm, dst_hbm, zeros_hbm, out_hbm,
               sidx, didx, rows, acc, gsems, ssems):
    c = lax.axis_index("c")
    s = lax.axis_index("s")
    wid = c * NS + s

    pltpu.sync_copy(zeros_hbm, acc.at[pl.ds(s * ROWS_PW, ROWS_PW)])

    @pl.when(s == 0)
    def _():
        pltpu.sync_copy(zeros_hbm.at[pl.ds(0, ROWS_TAIL)],
                        acc.at[pl.ds(NS * ROWS_PW, ROWS_TAIL)])

    plsc.subcore_barrier()

    def start(chunk, b):
        pltpu.sync_copy(src_hbm.at[wid, chunk], sidx.at[b])
        pltpu.sync_copy(dst_hbm.at[wid, chunk], didx.at[b])
        pltpu.async_copy(g_hbm.at[sidx.at[b]], rows.at[b], gsems[b])

    def finish(b):
        pltpu.make_async_copy(g_hbm.at[sidx.at[b]], rows.at[b],
                              gsems[b]).wait()
        pltpu.async_copy(rows.at[b], acc.at[didx.at[b]], ssems[b], add=True)

    def reuse(b):
        pltpu.make_async_copy(rows.at[b], acc.at[didx.at[b]], ssems[b]).wait()

    start(0, 0)

    @pl.loop(0, NCH - 1, step=2)
    def _(c0):
        for b in range(2):
            o = 1 - b
            if b == 0:
                @pl.when(c0 > 0)
                def _():
                    reuse(o)
            else:
                reuse(o)
            start(c0 + b + 1, o)
            finish(b)

    finish((NCH - 1) % 2)
    reuse(0)
    reuse(1)

    plsc.subcore_barrier()
    pltpu.sync_copy(acc.at[pl.ds(s * ROWS_PW, ROWS_PW)],
                    out_hbm.at[c, pl.ds(s * ROWS_PW, ROWS_PW)])

    @pl.when(s == 0)
    def _():
        pltpu.sync_copy(acc.at[pl.ds(NS * ROWS_PW, ROWS_TAIL)],
                        out_hbm.at[c, pl.ds(NS * ROWS_PW, ROWS_TAIL)])


@jax.jit
def _sc_propagate(g, src3, dst3, zeros_slice):
    return pl.kernel(
        _prop_body,
        out_type=jax.ShapeDtypeStruct((NC, N, D), jnp.float32),
        mesh=_sc_mesh,
        scratch_types=[
            pltpu.VMEM((2, K), jnp.int32),
            pltpu.VMEM((2, K), jnp.int32),
            pltpu.VMEM((2, K, D), jnp.float32),
            pltpu.VMEM_SHARED((N, D), jnp.float32),
            [pltpu.SemaphoreType.DMA] * 2,
            [pltpu.SemaphoreType.DMA] * 2,
        ],
    )(g, src3, dst3, zeros_slice)


def _deg_body(dst_hbm, zeros_hbm, out_hbm, didx, didxb, ones, acc, ssems):
    c = lax.axis_index("c")
    s = lax.axis_index("s")
    wid = c * NS + s

    @pl.loop(0, K)
    def _(i):
        ones[i] = jnp.full((16,), 1.0, jnp.float32)

    pltpu.sync_copy(zeros_hbm, acc.at[pl.ds(s * ROWS_PW, ROWS_PW)])

    @pl.when(s == 0)
    def _():
        pltpu.sync_copy(zeros_hbm.at[pl.ds(0, ROWS_TAIL)],
                        acc.at[pl.ds(NS * ROWS_PW, ROWS_TAIL)])

    pltpu.sync_copy(dst_hbm.at[wid], didx)
    plsc.subcore_barrier()

    def stage_and_fire(chunk, b):
        for j in range(K // 16):
            didxb[b, pl.ds(j * 16, 16)] = didx[chunk, pl.ds(j * 16, 16)]
        pltpu.async_copy(ones, acc.at[didxb.at[b]], ssems[b], add=True)

    def drain(b):
        pltpu.make_async_copy(ones, acc.at[didxb.at[b]], ssems[b]).wait()

    start_b = NB - 1
    for p in range(start_b):
        stage_and_fire(p, p)

    @pl.loop(0, NCH, step=NB)
    def _(c0):
        for b in range(NB):
            cc = c0 + b
            nxt = (b + start_b) % NB

            @pl.when(cc + start_b < NCH)
            def _():
                if b == 0:
                    @pl.when(cc >= 1)
                    def _():
                        drain(nxt)
                else:
                    drain(nxt)
                stage_and_fire(cc + start_b, nxt)

    for b in range(NB):
        drain(b)

    plsc.subcore_barrier()
    pltpu.sync_copy(acc.at[pl.ds(s * ROWS_PW, ROWS_PW)],
                    out_hbm.at[c, pl.ds(s * ROWS_PW, ROWS_PW)])

    @pl.when(s == 0)
    def _():
        pltpu.sync_copy(acc.at[pl.ds(NS * ROWS_PW, ROWS_TAIL)],
                        out_hbm.at[c, pl.ds(NS * ROWS_PW, ROWS_TAIL)])


@jax.jit
def _sc_degree(dst3, zeros16_slice):
    return pl.kernel(
        _deg_body,
        out_type=jax.ShapeDtypeStruct((NC, N, 16), jnp.float32),
        mesh=_sc_mesh,
        scratch_types=[
            pltpu.VMEM((NCH, K), jnp.int32),
            pltpu.VMEM((NB, K), jnp.int32),
            pltpu.VMEM((K, 16), jnp.float32),
            pltpu.VMEM_SHARED((N, 16), jnp.float32),
            [pltpu.SemaphoreType.DMA] * NB,
        ],
    )(dst3, zeros16_slice)


def _prep_body(degp_ref, x_ref, dinv_ref, g1_ref):
    deg = degp_ref[0, :, 0:1] + degp_ref[1, :, 0:1] + 1.0
    dinv = lax.rsqrt(deg)
    dinv_ref[...] = jnp.broadcast_to(dinv, (N, D))
    g1_ref[...] = x_ref[...] * dinv


@jax.jit
def _tc_prep(degp, x):
    return pl.pallas_call(
        _prep_body,
        out_shape=(jax.ShapeDtypeStruct((N, D), jnp.float32),
                   jax.ShapeDtypeStruct((N, D), jnp.float32)),
    )(degp, x)


def _mid_body(P_ref, g1_ref, dinv_ref, W1_ref, b1_ref, gamma_ref, beta_ref,
              a_ref, W2_ref, g2_ref):
    p1 = dinv_ref[...] * (P_ref[0] + P_ref[1] + g1_ref[...])
    h = jnp.dot(p1, W1_ref[...], preferred_element_type=jnp.float32)
    h = h + b1_ref[...]
    mean = jnp.mean(h, axis=0, keepdims=True)
    var = jnp.mean(h * h, axis=0, keepdims=True) - mean * mean
    xn = (h - mean) * lax.rsqrt(var + 1e-5) * gamma_ref[...] + beta_ref[...]
    xn = jnp.where(xn >= 0, xn, a_ref[0:1, 0:1] * xn)
    m = jnp.dot(xn, W2_ref[...], preferred_element_type=jnp.float32)
    g2_ref[...] = m * dinv_ref[...]


@jax.jit
def _tc_mid(P, g1, dinv, W1, b1, gamma, beta, a2d, W2):
    return pl.pallas_call(
        _mid_body,
        out_shape=jax.ShapeDtypeStruct((N, D), jnp.float32),
    )(P, g1, dinv, W1, b1, gamma, beta, a2d, W2)


def _final_body(Q_ref, g2_ref, dinv_ref, b2_ref, out_ref):
    out_ref[...] = (dinv_ref[...] * (Q_ref[0] + Q_ref[1] + g2_ref[...])
                    + b2_ref[...])


@jax.jit
def _tc_final(Q, g2, dinv, b2):
    return pl.pallas_call(
        _final_body,
        out_shape=jax.ShapeDtypeStruct((N, D), jnp.float32),
    )(Q, g2, dinv, b2)


@jax.jit
def kernel(x, edge_index, W1, b1, gamma, beta, a, W2, b2):
    src3 = edge_index[0].reshape(NC * NS, NCH, K)
    dst3 = edge_index[1].reshape(NC * NS, NCH, K)
    zeros128 = jnp.zeros((ROWS_PW, D), jnp.float32)
    zeros16 = jnp.zeros((ROWS_PW, 16), jnp.float32)
    a2d = jnp.broadcast_to(jnp.asarray(a, jnp.float32).reshape(1, 1), (8, 128))

    degp = _sc_degree(dst3, zeros16)
    dinv, g1 = _tc_prep(degp, x)
    P = _sc_propagate(g1, src3, dst3, zeros128)
    g2 = _tc_mid(P, g1, dinv, W1, b1.reshape(1, HID), gamma.reshape(1, HID),
                 beta.reshape(1, HID), a2d, W2)
    Q = _sc_propagate(g2, src3, dst3, zeros128)
    return _tc_final(Q, g2, dinv, b2.reshape(1, D))


# speedup vs baseline: 1.0024x; 1.0024x over previous
import functools

import jax
import jax.numpy as jnp
from jax import lax
from jax.experimental import pallas as pl
from jax.experimental.pallas import tpu as pltpu
from jax.experimental.pallas import tpu_sc as plsc

N = 10000
E = 320000
D = 128
HID = 256

NC = 2
NS = 16
EPW = E // (NC * NS)
ROWS_PW = 624
ROWS_TAIL = N - NS * ROWS_PW
K = 80
NCH = EPW // K
NB = 5

_sc_mesh = plsc.VectorSubcoreMesh(
    core_axis_name="c", subcore_axis_name="s", num_cores=NC, num_subcores=NS)


def _prop_body(g_hbm, src_hbm, dst_hbm, zeros_hbm, out_hbm,
               sidx, didx, rows, acc, gsems, ssems):
    c = lax.axis_index("c")
    s = lax.axis_index("s")
    wid = c * NS + s

    pltpu.sync_copy(zeros_hbm, acc.at[pl.ds(s * ROWS_PW, ROWS_PW)])

    @pl.when(s == 0)
    def _():
        pltpu.sync_copy(zeros_hbm.at[pl.ds(0, ROWS_TAIL)],
                        acc.at[pl.ds(NS * ROWS_PW, ROWS_TAIL)])

    plsc.subcore_barrier()

    def start(chunk, b):
        pltpu.sync_copy(src_hbm.at[wid, chunk], sidx.at[b])
        pltpu.sync_copy(dst_hbm.at[wid, chunk], didx.at[b])
        pltpu.async_copy(g_hbm.at[sidx.at[b]], rows.at[b], gsems[b])

    def finish(b):
        pltpu.make_async_copy(g_hbm.at[sidx.at[b]], rows.at[b],
                              gsems[b]).wait()
        pltpu.async_copy(rows.at[b], acc.at[didx.at[b]], ssems[b], add=True)

    def reuse(b):
        pltpu.make_async_copy(rows.at[b], acc.at[didx.at[b]], ssems[b]).wait()

    start(0, 0)
    start(1, 1)

    @pl.loop(0, NCH - 4, step=3)
    def _(c0):
        for b in range(3):
            cc = c0 + b
            o = (b + 2) % 3
            if b == 0:
                @pl.when(c0 > 0)
                def _():
                    reuse(o)
            else:
                reuse(o)
            start(cc + 2, o)
            finish(b)

    for t in (NCH - 2, NCH - 1):
        reuse((t + 2) % 3)
        finish(t % 3)
    reuse((NCH - 1) % 3)

    plsc.subcore_barrier()
    pltpu.sync_copy(acc.at[pl.ds(s * ROWS_PW, ROWS_PW)],
                    out_hbm.at[c, pl.ds(s * ROWS_PW, ROWS_PW)])

    @pl.when(s == 0)
    def _():
        pltpu.sync_copy(acc.at[pl.ds(NS * ROWS_PW, ROWS_TAIL)],
                        out_hbm.at[c, pl.ds(NS * ROWS_PW, ROWS_TAIL)])


@jax.jit
def _sc_propagate(g, src3, dst3, zeros_slice):
    return pl.kernel(
        _prop_body,
        out_type=jax.ShapeDtypeStruct((NC, N, D), jnp.float32),
        mesh=_sc_mesh,
        scratch_types=[
            pltpu.VMEM((3, K), jnp.int32),
            pltpu.VMEM((3, K), jnp.int32),
            pltpu.VMEM((3, K, D), jnp.float32),
            pltpu.VMEM_SHARED((N, D), jnp.float32),
            [pltpu.SemaphoreType.DMA] * 3,
            [pltpu.SemaphoreType.DMA] * 3,
        ],
    )(g, src3, dst3, zeros_slice)


def _deg_body(dst_hbm, zeros_hbm, out_hbm, didx, didxb, ones, acc, ssems):
    c = lax.axis_index("c")
    s = lax.axis_index("s")
    wid = c * NS + s

    @pl.loop(0, K)
    def _(i):
        ones[i] = jnp.full((16,), 1.0, jnp.float32)

    pltpu.sync_copy(zeros_hbm, acc.at[pl.ds(s * ROWS_PW, ROWS_PW)])

    @pl.when(s == 0)
    def _():
        pltpu.sync_copy(zeros_hbm.at[pl.ds(0, ROWS_TAIL)],
                        acc.at[pl.ds(NS * ROWS_PW, ROWS_TAIL)])

    pltpu.sync_copy(dst_hbm.at[wid], didx)
    plsc.subcore_barrier()

    def stage_and_fire(chunk, b):
        for j in range(K // 16):
            didxb[b, pl.ds(j * 16, 16)] = didx[chunk, pl.ds(j * 16, 16)]
        pltpu.async_copy(ones, acc.at[didxb.at[b]], ssems[b], add=True)

    def drain(b):
        pltpu.make_async_copy(ones, acc.at[didxb.at[b]], ssems[b]).wait()

    start_b = NB - 1
    for p in range(start_b):
        stage_and_fire(p, p)

    @pl.loop(0, NCH, step=NB)
    def _(c0):
        for b in range(NB):
            cc = c0 + b
            nxt = (b + start_b) % NB

            @pl.when(cc + start_b < NCH)
            def _():
                if b == 0:
                    @pl.when(cc >= 1)
                    def _():
                        drain(nxt)
                else:
                    drain(nxt)
                stage_and_fire(cc + start_b, nxt)

    for b in range(NB):
        drain(b)

    plsc.subcore_barrier()
    pltpu.sync_copy(acc.at[pl.ds(s * ROWS_PW, ROWS_PW)],
                    out_hbm.at[c, pl.ds(s * ROWS_PW, ROWS_PW)])

    @pl.when(s == 0)
    def _():
        pltpu.sync_copy(acc.at[pl.ds(NS * ROWS_PW, ROWS_TAIL)],
                        out_hbm.at[c, pl.ds(NS * ROWS_PW, ROWS_TAIL)])


@jax.jit
def _sc_degree(dst3, zeros16_slice):
    return pl.kernel(
        _deg_body,
        out_type=jax.ShapeDtypeStruct((NC, N, 16), jnp.float32),
        mesh=_sc_mesh,
        scratch_types=[
            pltpu.VMEM((NCH, K), jnp.int32),
            pltpu.VMEM((NB, K), jnp.int32),
            pltpu.VMEM((K, 16), jnp.float32),
            pltpu.VMEM_SHARED((N, 16), jnp.float32),
            [pltpu.SemaphoreType.DMA] * NB,
        ],
    )(dst3, zeros16_slice)


def _prep_body(degp_ref, x_ref, dinv_ref, g1_ref):
    deg = degp_ref[0, :, 0:1] + degp_ref[1, :, 0:1] + 1.0
    dinv = lax.rsqrt(deg)
    dinv_ref[...] = jnp.broadcast_to(dinv, (N, D))
    g1_ref[...] = x_ref[...] * dinv


@jax.jit
def _tc_prep(degp, x):
    return pl.pallas_call(
        _prep_body,
        out_shape=(jax.ShapeDtypeStruct((N, D), jnp.float32),
                   jax.ShapeDtypeStruct((N, D), jnp.float32)),
    )(degp, x)


def _mid_body(P_ref, g1_ref, dinv_ref, W1_ref, b1_ref, gamma_ref, beta_ref,
              a_ref, W2_ref, g2_ref):
    p1 = dinv_ref[...] * (P_ref[0] + P_ref[1] + g1_ref[...])
    h = jnp.dot(p1, W1_ref[...], preferred_element_type=jnp.float32)
    h = h + b1_ref[...]
    mean = jnp.mean(h, axis=0, keepdims=True)
    var = jnp.mean(h * h, axis=0, keepdims=True) - mean * mean
    xn = (h - mean) * lax.rsqrt(var + 1e-5) * gamma_ref[...] + beta_ref[...]
    xn = jnp.where(xn >= 0, xn, a_ref[0:1, 0:1] * xn)
    m = jnp.dot(xn, W2_ref[...], preferred_element_type=jnp.float32)
    g2_ref[...] = m * dinv_ref[...]


@jax.jit
def _tc_mid(P, g1, dinv, W1, b1, gamma, beta, a2d, W2):
    return pl.pallas_call(
        _mid_body,
        out_shape=jax.ShapeDtypeStruct((N, D), jnp.float32),
    )(P, g1, dinv, W1, b1, gamma, beta, a2d, W2)


def _final_body(Q_ref, g2_ref, dinv_ref, b2_ref, out_ref):
    out_ref[...] = (dinv_ref[...] * (Q_ref[0] + Q_ref[1] + g2_ref[...])
                    + b2_ref[...])


@jax.jit
def _tc_final(Q, g2, dinv, b2):
    return pl.pallas_call(
        _final_body,
        out_shape=jax.ShapeDtypeStruct((N, D), jnp.float32),
    )(Q, g2, dinv, b2)


@jax.jit
def kernel(x, edge_index, W1, b1, gamma, beta, a, W2, b2):
    src3 = edge_index[0].reshape(NC * NS, NCH, K)
    dst3 = edge_index[1].reshape(NC * NS, NCH, K)
    zeros128 = jnp.zeros((ROWS_PW, D), jnp.float32)
    zeros16 = jnp.zeros((ROWS_PW, 16), jnp.float32)
    a2d = jnp.broadcast_to(jnp.asarray(a, jnp.float32).reshape(1, 1), (8, 128))

    degp = _sc_degree(dst3, zeros16)
    dinv, g1 = _tc_prep(degp, x)
    P = _sc_propagate(g1, src3, dst3, zeros128)
    g2 = _tc_mid(P, g1, dinv, W1, b1.reshape(1, HID), gamma.reshape(1, HID),
                 beta.reshape(1, HID), a2d, W2)
    Q = _sc_propagate(g2, src3, dst3, zeros128)
    return _tc_final(Q, g2, dinv, b2.reshape(1, D))


# speedup vs baseline: 1.2200x; 1.2170x over previous
import functools

import jax
import jax.numpy as jnp
from jax import lax
from jax.experimental import pallas as pl
from jax.experimental.pallas import tpu as pltpu
from jax.experimental.pallas import tpu_sc as plsc

N = 10000
E = 320000
D = 128
HID = 256

NC = 2
NS = 16
EPW = E // (NC * NS)
ROWS_PW = 624
ROWS_TAIL = N - NS * ROWS_PW
K = 80
NCH = EPW // K
NB = 5

_sc_mesh = plsc.VectorSubcoreMesh(
    core_axis_name="c", subcore_axis_name="s", num_cores=NC, num_subcores=NS)


def _prop_body(g_hbm, src_hbm, dst_hbm, zeros_hbm, out_hbm,
               sidx, didx_full, didxb, rows, acc, gsems, ssems):
    c = lax.axis_index("c")
    s = lax.axis_index("s")
    wid = c * NS + s

    pltpu.sync_copy(zeros_hbm, acc.at[pl.ds(s * ROWS_PW, ROWS_PW)])

    @pl.when(s == 0)
    def _():
        pltpu.sync_copy(zeros_hbm.at[pl.ds(0, ROWS_TAIL)],
                        acc.at[pl.ds(NS * ROWS_PW, ROWS_TAIL)])

    pltpu.sync_copy(dst_hbm.at[wid], didx_full)
    plsc.subcore_barrier()

    def start(chunk, b):
        pltpu.sync_copy(src_hbm.at[wid, chunk], sidx.at[b])
        for j in range(K // 16):
            didxb[b, pl.ds(j * 16, 16)] = didx_full[chunk, pl.ds(j * 16, 16)]
        pltpu.async_copy(g_hbm.at[sidx.at[b]], rows.at[b], gsems[b])

    def finish(b):
        pltpu.make_async_copy(g_hbm.at[sidx.at[b]], rows.at[b],
                              gsems[b]).wait()
        pltpu.async_copy(rows.at[b], acc.at[didxb.at[b]], ssems[b], add=True)

    def reuse(b):
        pltpu.make_async_copy(rows.at[b], acc.at[didxb.at[b]], ssems[b]).wait()

    start(0, 0)

    @pl.loop(0, NCH - 1, step=2)
    def _(c0):
        for b in range(2):
            o = 1 - b
            if b == 0:
                @pl.when(c0 > 0)
                def _():
                    reuse(o)
            else:
                reuse(o)
            start(c0 + b + 1, o)
            finish(b)

    finish((NCH - 1) % 2)
    reuse(0)
    reuse(1)

    plsc.subcore_barrier()
    pltpu.sync_copy(acc.at[pl.ds(s * ROWS_PW, ROWS_PW)],
                    out_hbm.at[c, pl.ds(s * ROWS_PW, ROWS_PW)])

    @pl.when(s == 0)
    def _():
        pltpu.sync_copy(acc.at[pl.ds(NS * ROWS_PW, ROWS_TAIL)],
                        out_hbm.at[c, pl.ds(NS * ROWS_PW, ROWS_TAIL)])


@jax.jit
def _sc_propagate(g, src3, dst3, zeros_slice):
    return pl.kernel(
        _prop_body,
        out_type=jax.ShapeDtypeStruct((NC, N, D), jnp.float32),
        mesh=_sc_mesh,
        scratch_types=[
            pltpu.VMEM((2, K), jnp.int32),
            pltpu.VMEM((NCH, K), jnp.int32),
            pltpu.VMEM((2, K), jnp.int32),
            pltpu.VMEM((2, K, D), jnp.float32),
            pltpu.VMEM_SHARED((N, D), jnp.float32),
            [pltpu.SemaphoreType.DMA] * 2,
            [pltpu.SemaphoreType.DMA] * 2,
        ],
    )(g, src3, dst3, zeros_slice)


def _deg_body(dst_hbm, zeros_hbm, out_hbm, didx, didxb, ones, acc, ssems):
    c = lax.axis_index("c")
    s = lax.axis_index("s")
    wid = c * NS + s

    @pl.loop(0, K)
    def _(i):
        ones[i] = jnp.full((16,), 1.0, jnp.float32)

    pltpu.sync_copy(zeros_hbm, acc.at[pl.ds(s * ROWS_PW, ROWS_PW)])

    @pl.when(s == 0)
    def _():
        pltpu.sync_copy(zeros_hbm.at[pl.ds(0, ROWS_TAIL)],
                        acc.at[pl.ds(NS * ROWS_PW, ROWS_TAIL)])

    pltpu.sync_copy(dst_hbm.at[wid], didx)
    plsc.subcore_barrier()

    def stage_and_fire(chunk, b):
        for j in range(K // 16):
            didxb[b, pl.ds(j * 16, 16)] = didx[chunk, pl.ds(j * 16, 16)]
        pltpu.async_copy(ones, acc.at[didxb.at[b]], ssems[b], add=True)

    def drain(b):
        pltpu.make_async_copy(ones, acc.at[didxb.at[b]], ssems[b]).wait()

    start_b = NB - 1
    for p in range(start_b):
        stage_and_fire(p, p)

    @pl.loop(0, NCH, step=NB)
    def _(c0):
        for b in range(NB):
            cc = c0 + b
            nxt = (b + start_b) % NB

            @pl.when(cc + start_b < NCH)
            def _():
                if b == 0:
                    @pl.when(cc >= 1)
                    def _():
                        drain(nxt)
                else:
                    drain(nxt)
                stage_and_fire(cc + start_b, nxt)

    for b in range(NB):
        drain(b)

    plsc.subcore_barrier()
    pltpu.sync_copy(acc.at[pl.ds(s * ROWS_PW, ROWS_PW)],
                    out_hbm.at[c, pl.ds(s * ROWS_PW, ROWS_PW)])

    @pl.when(s == 0)
    def _():
        pltpu.sync_copy(acc.at[pl.ds(NS * ROWS_PW, ROWS_TAIL)],
                        out_hbm.at[c, pl.ds(NS * ROWS_PW, ROWS_TAIL)])


@jax.jit
def _sc_degree(dst3, zeros16_slice):
    return pl.kernel(
        _deg_body,
        out_type=jax.ShapeDtypeStruct((NC, N, 16), jnp.float32),
        mesh=_sc_mesh,
        scratch_types=[
            pltpu.VMEM((NCH, K), jnp.int32),
            pltpu.VMEM((NB, K), jnp.int32),
            pltpu.VMEM((K, 16), jnp.float32),
            pltpu.VMEM_SHARED((N, 16), jnp.float32),
            [pltpu.SemaphoreType.DMA] * NB,
        ],
    )(dst3, zeros16_slice)


def _prep_body(degp_ref, x_ref, dinv_ref, g1_ref):
    deg = degp_ref[0, :, 0:1] + degp_ref[1, :, 0:1] + 1.0
    dinv = lax.rsqrt(deg)
    dinv_ref[...] = jnp.broadcast_to(dinv, (N, D))
    g1_ref[...] = x_ref[...] * dinv


@jax.jit
def _tc_prep(degp, x):
    return pl.pallas_call(
        _prep_body,
        out_shape=(jax.ShapeDtypeStruct((N, D), jnp.float32),
                   jax.ShapeDtypeStruct((N, D), jnp.float32)),
    )(degp, x)


def _mid_body(P_ref, g1_ref, dinv_ref, W1_ref, b1_ref, gamma_ref, beta_ref,
              a_ref, W2_ref, g2_ref):
    p1 = dinv_ref[...] * (P_ref[0] + P_ref[1] + g1_ref[...])
    h = jnp.dot(p1, W1_ref[...], preferred_element_type=jnp.float32)
    h = h + b1_ref[...]
    mean = jnp.mean(h, axis=0, keepdims=True)
    var = jnp.mean(h * h, axis=0, keepdims=True) - mean * mean
    xn = (h - mean) * lax.rsqrt(var + 1e-5) * gamma_ref[...] + beta_ref[...]
    xn = jnp.where(xn >= 0, xn, a_ref[0:1, 0:1] * xn)
    m = jnp.dot(xn, W2_ref[...], preferred_element_type=jnp.float32)
    g2_ref[...] = m * dinv_ref[...]


@jax.jit
def _tc_mid(P, g1, dinv, W1, b1, gamma, beta, a2d, W2):
    return pl.pallas_call(
        _mid_body,
        out_shape=jax.ShapeDtypeStruct((N, D), jnp.float32),
    )(P, g1, dinv, W1, b1, gamma, beta, a2d, W2)


def _final_body(Q_ref, g2_ref, dinv_ref, b2_ref, out_ref):
    out_ref[...] = (dinv_ref[...] * (Q_ref[0] + Q_ref[1] + g2_ref[...])
                    + b2_ref[...])


@jax.jit
def _tc_final(Q, g2, dinv, b2):
    return pl.pallas_call(
        _final_body,
        out_shape=jax.ShapeDtypeStruct((N, D), jnp.float32),
    )(Q, g2, dinv, b2)


@jax.jit
def kernel(x, edge_index, W1, b1, gamma, beta, a, W2, b2):
    src3 = edge_index[0].reshape(NC * NS, NCH, K)
    dst3 = edge_index[1].reshape(NC * NS, NCH, K)
    zeros128 = jnp.zeros((ROWS_PW, D), jnp.float32)
    zeros16 = jnp.zeros((ROWS_PW, 16), jnp.float32)
    a2d = jnp.broadcast_to(jnp.asarray(a, jnp.float32).reshape(1, 1), (8, 128))

    degp = _sc_degree(dst3, zeros16)
    dinv, g1 = _tc_prep(degp, x)
    P = _sc_propagate(g1, src3, dst3, zeros128)
    g2 = _tc_mid(P, g1, dinv, W1, b1.reshape(1, HID), gamma.reshape(1, HID),
                 beta.reshape(1, HID), a2d, W2)
    Q = _sc_propagate(g2, src3, dst3, zeros128)
    return _tc_final(Q, g2, dinv, b2.reshape(1, D))
